# baseline (speedup 1.0000x reference)
import jax
import jax.numpy as jnp
import numpy as np
from jax import lax
from jax.experimental import pallas as pl
from jax.experimental.pallas import tpu as pltpu
from jax.experimental.pallas import tpu_sc as plsc

NC = 2
NS = 16
NW = NC * NS
LANES = 16

B, L = 4, 8192
N = B * L
TOK_PER_W = N // NW
T = 64
NCHUNK = TOK_PER_W // T
HALF = NCHUNK // 2
D = 256
DW = D // 2
NBLK = D // 32

RA = 2304
RA_W = RA // NW
RB = 128 * 128
RB_W = RB // NW
RB_CH = 128
REP_A = 1
REP_2 = NW

_IL = np.stack([np.arange(16), np.arange(16) + 16], axis=1).reshape(32)


def _prep_table(t, pad_rows=None):
    r = t.shape[0]
    bf = t.reshape(r, NBLK, 32)[:, :, _IL].reshape(r, D).astype(jnp.bfloat16)
    if pad_rows is not None and pad_rows > r:
        bf = jnp.concatenate(
            [bf, jnp.zeros((pad_rows - r, D), jnp.bfloat16)])
    return lax.bitcast_convert_type(
        bf.reshape(bf.shape[0], DW, 2), jnp.int32)


def _bsum(a, b):
    return plsc.bitcast(
        plsc.bitcast(a, jnp.bfloat16) + plsc.bitcast(b, jnp.bfloat16),
        jnp.int32)


def _build_body(src_t, dep_t, sp0_t, sp1_t, sp2_t,
                ca_hbm, cb_hbm, c2_hbm,
                src_v, dep_v, sp0_v, sp1_v, sp2_v, rb_a, rb_b,
                s0, s1, s2, s3, s4, soa, sob):
    wid = lax.axis_index("s") * NC + lax.axis_index("c")

    copies = (
        pltpu.make_async_copy(src_t, src_v, s0),
        pltpu.make_async_copy(dep_t, dep_v, s1),
        pltpu.make_async_copy(sp0_t, sp0_v, s2),
        pltpu.make_async_copy(sp1_t, sp1_v, s3),
        pltpu.make_async_copy(sp2_t, sp2_v, s4),
    )
    for dsc in copies:
        dsc.start()
    for dsc in copies:
        dsc.wait()

    c2 = pltpu.make_async_copy(
        sp2_v, c2_hbm.at[pl.ds(wid * 128, 128)], sob)
    c2.start()

    @plsc.parallel_loop(0, RA_W, unroll=4)
    def arow(k):
        r = wid * RA_W + k
        v = lax.shift_right_logical(r, 3)
        d = jnp.bitwise_and(r, 7)
        for blk in range(NBLK):
            sl = pl.ds(blk * LANES, LANES)
            rb_a[k, sl] = _bsum(src_v[v, sl], dep_v[d, sl])

    cas = [pltpu.make_async_copy(
        rb_a, ca_hbm.at[pl.ds(rep * RA + wid * RA_W, RA_W)], soa)
        for rep in range(REP_A)]
    for c in cas:
        c.start()

    for ch in range(RB_W // RB_CH):
        @plsc.parallel_loop(0, RB_CH, unroll=4)
        def brow(j):
            jj = ch * RB_CH + j
            i0 = wid * 4 + lax.shift_right_logical(jj, 7)
            i1 = jnp.bitwise_and(jj, 127)
            for blk in range(NBLK):
                sl = pl.ds(blk * LANES, LANES)
                rb_b[j, sl] = _bsum(sp0_v[i0, sl], sp1_v[i1, sl])

        pltpu.sync_copy(
            rb_b, cb_hbm.at[pl.ds(wid * RB_W + ch * RB_CH, RB_CH)])
    for c in cas:
        c.wait()
    c2.wait()


def _serve_body(vi, di, p0i, p1i, p2i, ca_t, cb_t, sp2_t,
                out_hbm,
                vi_v, di_v, p0_v, p1_v, p2_v, ia_v, ib_v,
                a0, a1, a2, b0, b1, b2, oa, ob,
                sa0, sa1, sa2, sb0, sb1, sb2, soa, sob):
    wid = lax.axis_index("s") * NC + lax.axis_index("c")
    base = wid * TOK_PER_W

    pltpu.sync_copy(vi.at[wid], vi_v)
    pltpu.sync_copy(di.at[wid], di_v)
    pltpu.sync_copy(p0i.at[wid], p0_v)
    pltpu.sync_copy(p1i.at[wid], p1_v)
    pltpu.sync_copy(p2i.at[wid], p2_v)

    ra_off = jnp.bitwise_and(wid, REP_A - 1) * RA
    r2_off = wid * 128

    def idxrow(c, carry):
        for g in range(T // LANES):
            sl = pl.ds(g * LANES, LANES)
            ia_v[c, sl] = vi_v[c, sl] * 8 + di_v[c, sl] + ra_off
            ib_v[c, sl] = p0_v[c, sl] * 128 + p1_v[c, sl]
            p2_v[c, sl] = p2_v[c, sl] + r2_off
        return carry

    lax.fori_loop(0, NCHUNK, idxrow, 0, unroll=False)

    sets = (
        ((a0, a1, a2), (sa0, sa1, sa2), oa, soa),
        ((b0, b1, b2), (sb0, sb1, sb2), ob, sob),
    )

    def gathers(c, p):
        bufs, sems, _, _ = sets[p]
        return (
            pltpu.make_async_copy(ca_t.at[ia_v.at[c]], bufs[0], sems[0]),
            pltpu.make_async_copy(cb_t.at[ib_v.at[c]], bufs[1], sems[1]),
            pltpu.make_async_copy(sp2_t.at[p2_v.at[c]], bufs[2], sems[2]),
        )

    def fire(c, p):
        for dsc in gathers(c, p):
            dsc.start()

    def wait_gathers(c, p):
        for dsc in gathers(c, p):
            dsc.wait()

    def out_copy(c, p):
        _, _, obuf, osem = sets[p]
        return pltpu.make_async_copy(
            obuf, out_hbm.at[pl.ds(base + c * T, T)], osem)

    hi16 = jnp.full((LANES,), -65536, dtype=jnp.int32)
    bf = jnp.bfloat16

    def process(c, p, k):
        bufs, _, obuf, _ = sets[p]
        wait_gathers(c, p)

        @pl.when(k > 0)
        def _():
            out_copy(c - 2, p).wait()

        g0, g1, g2 = bufs

        @plsc.parallel_loop(0, T, unroll=2)
        def row(r):
            for d in range(NBLK):
                sl = pl.ds(d * LANES, LANES)
                acc = (plsc.bitcast(g0[r, sl], bf)
                       + plsc.bitcast(g1[r, sl], bf)
                       ) + plsc.bitcast(g2[r, sl], bf)
                w = plsc.bitcast(acc, jnp.int32)
                even = lax.bitcast_convert_type(
                    jnp.left_shift(w, 16), jnp.float32)
                odd = lax.bitcast_convert_type(
                    jnp.bitwise_and(w, hi16), jnp.float32)
                obuf[r, pl.ds(d * 32, LANES)] = even
                obuf[r, pl.ds(d * 32 + LANES, LANES)] = odd
        out_copy(c, p).start()

    fire(0, 0)

    def pair(k, carry):
        c0 = 2 * k
        fire(c0 + 1, 1)
        process(c0, 0, k)

        @pl.when(k < HALF - 1)
        def _():
            fire(c0 + 2, 0)

        process(c0 + 1, 1, k)
        return carry

    lax.fori_loop(0, HALF, pair, 0, unroll=False)
    out_copy(NCHUNK - 2, 0).wait()
    out_copy(NCHUNK - 1, 1).wait()


@jax.jit
def _embed_sum(vi, di, p0i, p1i, p2i, src_t, dep_t, sp0_t, sp1_t, sp2_t):
    mesh = plsc.VectorSubcoreMesh(
        core_axis_name="c", subcore_axis_name="s",
        num_cores=NC, num_subcores=NS)
    params = pltpu.CompilerParams(needs_layout_passes=False)

    build = pl.kernel(
        _build_body,
        out_type=(jax.ShapeDtypeStruct((REP_A * RA, DW), jnp.int32),
                  jax.ShapeDtypeStruct((RB, DW), jnp.int32),
                  jax.ShapeDtypeStruct((REP_2 * 128, DW), jnp.int32)),
        mesh=mesh,
        compiler_params=params,
        scratch_types=(
            [pltpu.VMEM((RA // 8, DW), jnp.int32),
             pltpu.VMEM((8, DW), jnp.int32),
             pltpu.VMEM((128, DW), jnp.int32),
             pltpu.VMEM((128, DW), jnp.int32),
             pltpu.VMEM((128, DW), jnp.int32),
             pltpu.VMEM((RA_W, DW), jnp.int32),
             pltpu.VMEM((RB_CH, DW), jnp.int32)]
            + [pltpu.SemaphoreType.DMA] * 7
        ),
    )
    ca_t, cb_t, c2_t = build(src_t, dep_t, sp0_t, sp1_t, sp2_t)

    serve = pl.kernel(
        _serve_body,
        out_type=jax.ShapeDtypeStruct((N, D), jnp.float32),
        mesh=mesh,
        compiler_params=params,
        scratch_types=(
            [pltpu.VMEM((NCHUNK, T), jnp.int32)] * 7
            + [pltpu.VMEM((T, DW), jnp.int32)] * 6
            + [pltpu.VMEM((T, D), jnp.float32)] * 2
            + [pltpu.SemaphoreType.DMA] * 8
        ),
    )
    return serve(vi, di, p0i, p1i, p2i, ca_t, cb_t, c2_t)


def kernel(value, depth, position, src_table, depth_table, sp_table0,
           sp_table1, sp_table2):
    shp = (NW, NCHUNK, T)
    vi = value.reshape(shp).astype(jnp.int32)
    di = depth.reshape(shp).astype(jnp.int32)
    p0i = position[:, :, 0].reshape(shp).astype(jnp.int32)
    p1i = position[:, :, 1].reshape(shp).astype(jnp.int32)
    p2i = position[:, :, 2].reshape(shp).astype(jnp.int32)
    out = _embed_sum(vi, di, p0i, p1i, p2i,
                     _prep_table(src_table, pad_rows=RA // 8),
                     _prep_table(depth_table, pad_rows=8),
                     _prep_table(sp_table0), _prep_table(sp_table1),
                     _prep_table(sp_table2))
    return out.reshape(B, L, D)


# speedup vs baseline: 1.0229x; 1.0229x over previous
import jax
import jax.numpy as jnp
import numpy as np
from jax import lax
from jax.experimental import pallas as pl
from jax.experimental.pallas import tpu as pltpu
from jax.experimental.pallas import tpu_sc as plsc

NC = 2
NS = 16
NW = NC * NS
LANES = 16

B, L = 4, 8192
N = B * L
TOK_PER_W = N // NW
T = 64
NCHUNK = TOK_PER_W // T
HALF = NCHUNK // 2
D = 256
DW = D // 2
NBLK = D // 32

RA = 2304
RA_W = RA // NW
RB = 128 * 128
RB_W = RB // NW
RB_CH = 128
REP_A = 2
REP_2 = NW

_IL = np.stack([np.arange(16), np.arange(16) + 16], axis=1).reshape(32)


def _prep_table(t, pad_rows=None):
    r = t.shape[0]
    bf = t.reshape(r, NBLK, 32)[:, :, _IL].reshape(r, D).astype(jnp.bfloat16)
    if pad_rows is not None and pad_rows > r:
        bf = jnp.concatenate(
            [bf, jnp.zeros((pad_rows - r, D), jnp.bfloat16)])
    return lax.bitcast_convert_type(
        bf.reshape(bf.shape[0], DW, 2), jnp.int32)


def _bsum(a, b):
    return plsc.bitcast(
        plsc.bitcast(a, jnp.bfloat16) + plsc.bitcast(b, jnp.bfloat16),
        jnp.int32)


def _build_body(src_t, dep_t, sp0_t, sp1_t, sp2_t,
                ca_hbm, cb_hbm, c2_hbm,
                src_v, dep_v, sp0_v, sp1_v, sp2_v, rb_a, rb_b,
                s0, s1, s2, s3, s4, soa, sob):
    wid = lax.axis_index("s") * NC + lax.axis_index("c")

    copies = (
        pltpu.make_async_copy(src_t, src_v, s0),
        pltpu.make_async_copy(dep_t, dep_v, s1),
        pltpu.make_async_copy(sp0_t, sp0_v, s2),
        pltpu.make_async_copy(sp1_t, sp1_v, s3),
        pltpu.make_async_copy(sp2_t, sp2_v, s4),
    )
    for dsc in copies:
        dsc.start()
    for dsc in copies:
        dsc.wait()

    c2 = pltpu.make_async_copy(
        sp2_v, c2_hbm.at[pl.ds(wid * 128, 128)], sob)
    c2.start()

    @plsc.parallel_loop(0, RA_W, unroll=4)
    def arow(k):
        r = wid * RA_W + k
        v = lax.shift_right_logical(r, 3)
        d = jnp.bitwise_and(r, 7)
        for blk in range(NBLK):
            sl = pl.ds(blk * LANES, LANES)
            rb_a[k, sl] = _bsum(src_v[v, sl], dep_v[d, sl])

    cas = [pltpu.make_async_copy(
        rb_a, ca_hbm.at[pl.ds(rep * RA + wid * RA_W, RA_W)], soa)
        for rep in range(REP_A)]
    for c in cas:
        c.start()

    for ch in range(RB_W // RB_CH):
        @plsc.parallel_loop(0, RB_CH, unroll=4)
        def brow(j):
            jj = ch * RB_CH + j
            i0 = wid * 4 + lax.shift_right_logical(jj, 7)
            i1 = jnp.bitwise_and(jj, 127)
            for blk in range(NBLK):
                sl = pl.ds(blk * LANES, LANES)
                rb_b[j, sl] = _bsum(sp0_v[i0, sl], sp1_v[i1, sl])

        pltpu.sync_copy(
            rb_b, cb_hbm.at[pl.ds(wid * RB_W + ch * RB_CH, RB_CH)])
    for c in cas:
        c.wait()
    c2.wait()


def _serve_body(vi, di, p0i, p1i, p2i, ca_t, cb_t, sp2_t,
                out_hbm,
                vi_v, di_v, p0_v, p1_v, p2_v, ia_v, ib_v,
                a0, a1, a2, b0, b1, b2, oa, ob,
                sa0, sa1, sa2, sb0, sb1, sb2, soa, sob):
    wid = lax.axis_index("s") * NC + lax.axis_index("c")
    base = wid * TOK_PER_W

    pltpu.sync_copy(vi.at[wid], vi_v)
    pltpu.sync_copy(di.at[wid], di_v)
    pltpu.sync_copy(p0i.at[wid], p0_v)
    pltpu.sync_copy(p1i.at[wid], p1_v)
    pltpu.sync_copy(p2i.at[wid], p2_v)

    ra_off = jnp.bitwise_and(wid, REP_A - 1) * RA
    r2_off = wid * 128

    def idxrow(c, carry):
        for g in range(T // LANES):
            sl = pl.ds(g * LANES, LANES)
            ia_v[c, sl] = vi_v[c, sl] * 8 + di_v[c, sl] + ra_off
            ib_v[c, sl] = p0_v[c, sl] * 128 + p1_v[c, sl]
            p2_v[c, sl] = p2_v[c, sl] + r2_off
        return carry

    lax.fori_loop(0, NCHUNK, idxrow, 0, unroll=False)

    sets = (
        ((a0, a1, a2), (sa0, sa1, sa2), oa, soa),
        ((b0, b1, b2), (sb0, sb1, sb2), ob, sob),
    )

    def gathers(c, p):
        bufs, sems, _, _ = sets[p]
        return (
            pltpu.make_async_copy(ca_t.at[ia_v.at[c]], bufs[0], sems[0]),
            pltpu.make_async_copy(cb_t.at[ib_v.at[c]], bufs[1], sems[1]),
            pltpu.make_async_copy(sp2_t.at[p2_v.at[c]], bufs[2], sems[2]),
        )

    def fire(c, p):
        for dsc in gathers(c, p):
            dsc.start()

    def wait_gathers(c, p):
        for dsc in gathers(c, p):
            dsc.wait()

    def out_copy(c, p):
        _, _, obuf, osem = sets[p]
        return pltpu.make_async_copy(
            obuf, out_hbm.at[pl.ds(base + c * T, T)], osem)

    hi16 = jnp.full((LANES,), -65536, dtype=jnp.int32)
    bf = jnp.bfloat16

    def process(c, p, k):
        bufs, _, obuf, _ = sets[p]
        wait_gathers(c, p)

        @pl.when(k > 0)
        def _():
            out_copy(c - 2, p).wait()

        g0, g1, g2 = bufs

        @plsc.parallel_loop(0, T, unroll=2)
        def row(r):
            for d in range(NBLK):
                sl = pl.ds(d * LANES, LANES)
                acc = (plsc.bitcast(g0[r, sl], bf)
                       + plsc.bitcast(g1[r, sl], bf)
                       ) + plsc.bitcast(g2[r, sl], bf)
                w = plsc.bitcast(acc, jnp.int32)
                even = lax.bitcast_convert_type(
                    jnp.left_shift(w, 16), jnp.float32)
                odd = lax.bitcast_convert_type(
                    jnp.bitwise_and(w, hi16), jnp.float32)
                obuf[r, pl.ds(d * 32, LANES)] = even
                obuf[r, pl.ds(d * 32 + LANES, LANES)] = odd
        out_copy(c, p).start()

    fire(0, 0)

    def pair(k, carry):
        c0 = 2 * k
        fire(c0 + 1, 1)
        process(c0, 0, k)

        @pl.when(k < HALF - 1)
        def _():
            fire(c0 + 2, 0)

        process(c0 + 1, 1, k)
        return carry

    lax.fori_loop(0, HALF, pair, 0, unroll=False)
    out_copy(NCHUNK - 2, 0).wait()
    out_copy(NCHUNK - 1, 1).wait()


@jax.jit
def _embed_sum(vi, di, p0i, p1i, p2i, src_t, dep_t, sp0_t, sp1_t, sp2_t):
    mesh = plsc.VectorSubcoreMesh(
        core_axis_name="c", subcore_axis_name="s",
        num_cores=NC, num_subcores=NS)
    params = pltpu.CompilerParams(needs_layout_passes=False)

    build = pl.kernel(
        _build_body,
        out_type=(jax.ShapeDtypeStruct((REP_A * RA, DW), jnp.int32),
                  jax.ShapeDtypeStruct((RB, DW), jnp.int32),
                  jax.ShapeDtypeStruct((REP_2 * 128, DW), jnp.int32)),
        mesh=mesh,
        compiler_params=params,
        scratch_types=(
            [pltpu.VMEM((RA // 8, DW), jnp.int32),
             pltpu.VMEM((8, DW), jnp.int32),
             pltpu.VMEM((128, DW), jnp.int32),
             pltpu.VMEM((128, DW), jnp.int32),
             pltpu.VMEM((128, DW), jnp.int32),
             pltpu.VMEM((RA_W, DW), jnp.int32),
             pltpu.VMEM((RB_CH, DW), jnp.int32)]
            + [pltpu.SemaphoreType.DMA] * 7
        ),
    )
    ca_t, cb_t, c2_t = build(src_t, dep_t, sp0_t, sp1_t, sp2_t)

    serve = pl.kernel(
        _serve_body,
        out_type=jax.ShapeDtypeStruct((N, D), jnp.float32),
        mesh=mesh,
        compiler_params=params,
        scratch_types=(
            [pltpu.VMEM((NCHUNK, T), jnp.int32)] * 7
            + [pltpu.VMEM((T, DW), jnp.int32)] * 6
            + [pltpu.VMEM((T, D), jnp.float32)] * 2
            + [pltpu.SemaphoreType.DMA] * 8
        ),
    )
    return serve(vi, di, p0i, p1i, p2i, ca_t, cb_t, c2_t)


def kernel(value, depth, position, src_table, depth_table, sp_table0,
           sp_table1, sp_table2):
    shp = (NW, NCHUNK, T)
    vi = value.reshape(shp).astype(jnp.int32)
    di = depth.reshape(shp).astype(jnp.int32)
    p0i = position[:, :, 0].reshape(shp).astype(jnp.int32)
    p1i = position[:, :, 1].reshape(shp).astype(jnp.int32)
    p2i = position[:, :, 2].reshape(shp).astype(jnp.int32)
    out = _embed_sum(vi, di, p0i, p1i, p2i,
                     _prep_table(src_table, pad_rows=RA // 8),
                     _prep_table(depth_table, pad_rows=8),
                     _prep_table(sp_table0), _prep_table(sp_table1),
                     _prep_table(sp_table2))
    return out.reshape(B, L, D)
